# baseline (speedup 1.0000x reference)
import functools

import jax
import jax.numpy as jnp
from jax import lax
from jax.experimental import pallas as pl
from jax.experimental.pallas import tpu as pltpu
from jax.experimental.pallas import tpu_sc as plsc

F32 = jnp.float32

NC = 2
NS = 16
NW = NC * NS

LW = 128

CW = 128


def _h0_body(x_ref, w_ref, b_ref, o_ref):
    nb = x_ref.shape[0]
    h = lax.dot_general(x_ref[...], w_ref[...], (((1,), (1,)), ((), ())),
                        preferred_element_type=F32)
    h = jnp.maximum(h + b_ref[...], 0.0)
    o_ref[...] = jnp.concatenate(
        [h, jnp.zeros((nb, LW - h.shape[1]), F32)], axis=1)


def _project(x, w_proj, b_proj2d, nb):
    n, _ = x.shape
    h = w_proj.shape[0]
    grid = n // nb
    return pl.pallas_call(
        _h0_body,
        grid=(grid,),
        in_specs=[
            pl.BlockSpec((nb, x.shape[1]), lambda i: (i, 0)),
            pl.BlockSpec(w_proj.shape, lambda i: (0, 0)),
            pl.BlockSpec((1, h), lambda i: (0, 0)),
        ],
        out_specs=pl.BlockSpec((nb, LW), lambda i: (i, 0)),
        out_shape=jax.ShapeDtypeStruct((n, LW), F32),
    )(x, w_proj, b_proj2d)


def _msg_body(hs_ref, ea_ref, s_ref, t_ref, wk_ref, wb_ref, o_ref):
    h = hs_ref[:, 0:wk_ref.shape[1]]
    a = ea_ref[...]
    eb = h.shape[0]

    def dot(lhs, w):
        return lax.dot_general(lhs, w, (((1,), (0,)), ((), ())),
                               preferred_element_type=F32)

    v = dot(a, s_ref[...]) * dot(h, t_ref[...])
    m = dot(v, wk_ref[...]) + dot(h, wb_ref[...])
    o_ref[...] = jnp.concatenate(
        [m, jnp.zeros((eb, LW - m.shape[1]), F32)], axis=1)


def _messages(hsrc, edge_attr, s_mat, t_mat, w_k, w_b, eb):
    e = hsrc.shape[0]
    h = w_k.shape[1]
    grid = e // eb
    return pl.pallas_call(
        _msg_body,
        grid=(grid,),
        in_specs=[
            pl.BlockSpec((eb, LW), lambda i: (i, 0)),
            pl.BlockSpec((eb, edge_attr.shape[1]), lambda i: (i, 0)),
            pl.BlockSpec(s_mat.shape, lambda i: (0, 0)),
            pl.BlockSpec(t_mat.shape, lambda i: (0, 0)),
            pl.BlockSpec(w_k.shape, lambda i: (0, 0)),
            pl.BlockSpec(w_b.shape, lambda i: (0, 0)),
        ],
        out_specs=pl.BlockSpec((eb, LW), lambda i: (i, 0)),
        out_shape=jax.ShapeDtypeStruct((e, LW), F32),
    )(hsrc, edge_attr, s_mat, t_mat, w_k, w_b)


def _gather_rows(node, src_b2d, src_a2d):
    n = node.shape[0]
    rows_b, cw = src_b2d.shape
    rows_a = src_a2d.shape[0]
    ch_b = rows_b // NW
    ch_a = rows_a // NW
    e_b = rows_b * cw
    e_a = rows_a * cw
    r_per = ((n + NS - 1) // NS + 7) // 8 * 8
    r_last = n - (NS - 1) * r_per
    mesh = plsc.VectorSubcoreMesh(core_axis_name="c", subcore_axis_name="s",
                                  num_cores=NC, num_subcores=NS)

    def body(node_hbm, srcb_hbm, srca_hbm, outb_hbm, outa_hbm, shared,
             idxb_v, idxa_v, row_a, row_b, sem_a, sem_b):
        sid = lax.axis_index("s")
        wid = sid * NC + lax.axis_index("c")

        @pl.when(sid < NS - 1)
        def _stage_main():
            pltpu.sync_copy(node_hbm.at[pl.ds(sid * r_per, r_per)],
                            shared.at[pl.ds(sid * r_per, r_per)])

        @pl.when(sid == NS - 1)
        def _stage_last():
            pltpu.sync_copy(node_hbm.at[pl.ds((NS - 1) * r_per, r_last)],
                            shared.at[pl.ds((NS - 1) * r_per, r_last)])

        plsc.subcore_barrier()
        pltpu.sync_copy(srcb_hbm.at[pl.ds(wid * ch_b, ch_b)], idxb_v)
        pltpu.sync_copy(srca_hbm.at[pl.ds(wid * ch_a, ch_a)], idxa_v)

        def run(idx_v, out_hbm, ch):
            def out_at(j):
                return out_hbm.at[pl.ds((wid * ch + j) * cw, cw)]

            pltpu.async_copy(shared.at[idx_v.at[0]], row_a, sem_a)

            def pair(t, carry):
                j0 = 2 * t
                pltpu.async_copy(shared.at[idx_v.at[j0 + 1]], row_b, sem_b)
                pltpu.make_async_copy(shared.at[idx_v.at[j0]], row_a,
                                      sem_a).wait()
                pltpu.sync_copy(row_a, out_at(j0))

                @pl.when(t + 1 < ch // 2)
                def _next():
                    pltpu.async_copy(shared.at[idx_v.at[j0 + 2]], row_a,
                                     sem_a)

                pltpu.make_async_copy(shared.at[idx_v.at[j0 + 1]], row_b,
                                      sem_b).wait()
                pltpu.sync_copy(row_b, out_at(j0 + 1))
                return carry

            lax.fori_loop(0, ch // 2, pair, 0)

        run(idxb_v, outb_hbm, ch_b)
        run(idxa_v, outa_hbm, ch_a)

    return pl.kernel(
        body,
        out_type=[jax.ShapeDtypeStruct((e_b, LW), F32),
                  jax.ShapeDtypeStruct((e_a, LW), F32)],
        mesh=mesh,
        scratch_types=[
            pltpu.VMEM_SHARED((n, LW), F32),
            pltpu.VMEM((ch_b, cw), jnp.int32),
            pltpu.VMEM((ch_a, cw), jnp.int32),
            pltpu.VMEM((cw, LW), F32),
            pltpu.VMEM((cw, LW), F32),
            pltpu.SemaphoreType.DMA,
            pltpu.SemaphoreType.DMA,
        ],
    )(node, src_b2d, src_a2d)


def _scatter_add(m, dst2d, zeros_nh):
    np_ = zeros_nh.shape[0]
    n = np_ - 8
    rows, cw = dst2d.shape
    ch = rows // NW
    r_per = ((n + NS - 1) // NS + 7) // 8 * 8
    r_last = n - (NS - 1) * r_per
    mesh = plsc.VectorSubcoreMesh(core_axis_name="c", subcore_axis_name="s",
                                  num_cores=NC, num_subcores=NS)

    def body(m_hbm, dst_hbm, z_hbm, out_hbm, shared, idx_v, row_a, row_b,
             sem_a, sem_b):
        cid = lax.axis_index("c")
        sid = lax.axis_index("s")
        wid = sid * NC + cid

        @pl.when(sid == 0)
        def _init():
            pltpu.sync_copy(z_hbm, shared)

        plsc.subcore_barrier()
        pltpu.sync_copy(dst_hbm.at[pl.ds(wid * ch, ch)], idx_v)

        def m_at(j):
            return m_hbm.at[pl.ds((wid * ch + j) * cw, cw)]

        pltpu.async_copy(m_at(0), row_a, sem_a)

        def pair(t, carry):
            j0 = 2 * t
            pltpu.async_copy(m_at(j0 + 1), row_b, sem_b)
            pltpu.make_async_copy(m_at(j0), row_a, sem_a).wait()
            pltpu.sync_copy(row_a, shared.at[idx_v.at[j0]], add=True)

            @pl.when(t + 1 < ch // 2)
            def _next():
                pltpu.async_copy(m_at(j0 + 2), row_a, sem_a)

            pltpu.make_async_copy(m_at(j0 + 1), row_b, sem_b).wait()
            pltpu.sync_copy(row_b, shared.at[idx_v.at[j0 + 1]], add=True)
            return carry

        lax.fori_loop(0, ch // 2, pair, 0)
        plsc.subcore_barrier()

        @pl.when(sid < NS - 1)
        def _copy_main():
            pltpu.sync_copy(shared.at[pl.ds(sid * r_per, r_per)],
                            out_hbm.at[pl.ds(cid * n + sid * r_per, r_per)])

        @pl.when(sid == NS - 1)
        def _copy_last():
            pltpu.sync_copy(
                shared.at[pl.ds((NS - 1) * r_per, r_last)],
                out_hbm.at[pl.ds(cid * n + (NS - 1) * r_per, r_last)])

    return pl.kernel(
        body,
        out_type=jax.ShapeDtypeStruct((2 * n, LW), F32),
        mesh=mesh,
        scratch_types=[
            pltpu.VMEM_SHARED((np_, LW), F32),
            pltpu.VMEM((ch, cw), jnp.int32),
            pltpu.VMEM((cw, LW), F32),
            pltpu.VMEM((cw, LW), F32),
            pltpu.SemaphoreType.DMA,
            pltpu.SemaphoreType.DMA,
        ],
    )(m, dst2d, zeros_nh)


def _gru_body(p0_ref, p1_ref, p2_ref, p3_ref, h_ref, wih_ref, whh_ref,
              bih_ref, bhh_ref, bc_ref, o_ref, *, h_dim):
    nb = h_ref.shape[0]
    hdim = h_dim
    a = jnp.maximum(p0_ref[:, 0:hdim] + p1_ref[:, 0:hdim] +
                    p2_ref[:, 0:hdim] + p3_ref[:, 0:hdim] + bc_ref[...], 0.0)
    hprev = h_ref[:, 0:hdim]

    def dot_t(lhs, w):
        return lax.dot_general(lhs, w, (((1,), (1,)), ((), ())),
                               preferred_element_type=F32)

    wih = wih_ref[...]
    whh = whh_ref[...]
    i_r = dot_t(a, wih[0:hdim]) + bih_ref[0:1]
    i_z = dot_t(a, wih[hdim:2 * hdim]) + bih_ref[1:2]
    i_n = dot_t(a, wih[2 * hdim:3 * hdim]) + bih_ref[2:3]
    h_r = dot_t(hprev, whh[0:hdim]) + bhh_ref[0:1]
    h_z = dot_t(hprev, whh[hdim:2 * hdim]) + bhh_ref[1:2]
    h_n = dot_t(hprev, whh[2 * hdim:3 * hdim]) + bhh_ref[2:3]
    r = jax.nn.sigmoid(i_r + h_r)
    z = jax.nn.sigmoid(i_z + h_z)
    nn = jnp.tanh(i_n + r * h_n)
    hnew = (1.0 - z) * nn + z * hprev
    o_ref[...] = jnp.concatenate(
        [hnew, jnp.zeros((nb, LW - hdim), F32)], axis=1)


def _gru(parts_a, parts_b, hidden, w_ih, w_hh, b_ih3, b_hh3, b_conv2d, nb):
    n = hidden.shape[0]
    h = w_ih.shape[1]
    grid = n // nb
    return pl.pallas_call(
        functools.partial(_gru_body, h_dim=h),
        grid=(grid,),
        in_specs=[
            pl.BlockSpec((nb, LW), lambda i: (i, 0)),
            pl.BlockSpec((nb, LW), lambda i, g=grid: (i + g, 0)),
            pl.BlockSpec((nb, LW), lambda i: (i, 0)),
            pl.BlockSpec((nb, LW), lambda i, g=grid: (i + g, 0)),
            pl.BlockSpec((nb, LW), lambda i: (i, 0)),
            pl.BlockSpec(w_ih.shape, lambda i: (0, 0)),
            pl.BlockSpec(w_hh.shape, lambda i: (0, 0)),
            pl.BlockSpec((3, h), lambda i: (0, 0)),
            pl.BlockSpec((3, h), lambda i: (0, 0)),
            pl.BlockSpec((1, h), lambda i: (0, 0)),
        ],
        out_specs=pl.BlockSpec((nb, LW), lambda i: (i, 0)),
        out_shape=jax.ShapeDtypeStruct((n, LW), F32),
    )(parts_a, parts_a, parts_b, parts_b, hidden, w_ih, w_hh, b_ih3, b_hh3,
      b_conv2d)


def _s2s_body(h0_ref, ht_ref, wli_ref, wlh_ref, bl_ref, wsp_ref, bsp_ref,
              pa_ref, o_ref, *, n_s2s, h_dim):
    na = jnp.concatenate([h0_ref[:, 0:h_dim], ht_ref[:, 0:h_dim]], axis=1)
    d = na.shape[1]
    wli = wli_ref[...]
    wlh = wlh_ref[...]
    bl = bl_ref[...]
    hs = jnp.zeros((1, d), dtype=F32)
    cs = jnp.zeros((1, d), dtype=F32)
    qs = jnp.zeros((1, 2 * d), dtype=F32)

    def dot_t(lhs, w):
        return lax.dot_general(lhs, w, (((1,), (1,)), ((), ())),
                               preferred_element_type=F32)

    for _ in range(n_s2s):
        gi = dot_t(qs, wli[0:d]) + dot_t(hs, wlh[0:d]) + bl[0:1]
        gf = dot_t(qs, wli[d:2 * d]) + dot_t(hs, wlh[d:2 * d]) + bl[1:2]
        gg = dot_t(qs, wli[2 * d:3 * d]) + dot_t(hs, wlh[2 * d:3 * d]) + bl[2:3]
        go = dot_t(qs, wli[3 * d:4 * d]) + dot_t(hs, wlh[3 * d:4 * d]) + bl[3:4]
        ig = jax.nn.sigmoid(gi)
        fg = jax.nn.sigmoid(gf)
        gg = jnp.tanh(gg)
        og = jax.nn.sigmoid(go)
        cs = fg * cs + ig * gg
        hs = og * jnp.tanh(cs)
        e = dot_t(na, hs)
        mx = jnp.max(e)
        ex = jnp.exp(e - mx)
        alpha = ex / jnp.sum(ex)
        r = lax.dot_general(alpha, na, (((0,), (0,)), ((), ())),
                            preferred_element_type=F32)
        qs = jnp.concatenate([hs, r], axis=1)

    out = dot_t(qs, wsp_ref[...]) + bsp_ref[...]
    o_ref[...] = jnp.where(out >= 0.0, out, pa_ref[...] * out)


def _set2set(h0, ht, w_li, w_lh, bl4, w_sp, b_sp2d, pa2d, n_s2s, h):
    n = h0.shape[0]
    ro = w_sp.shape[0]
    return pl.pallas_call(
        functools.partial(_s2s_body, n_s2s=n_s2s, h_dim=h),
        in_specs=[
            pl.BlockSpec((n, LW), lambda: (0, 0)),
            pl.BlockSpec((n, LW), lambda: (0, 0)),
            pl.BlockSpec(w_li.shape, lambda: (0, 0)),
            pl.BlockSpec(w_lh.shape, lambda: (0, 0)),
            pl.BlockSpec(bl4.shape, lambda: (0, 0)),
            pl.BlockSpec(w_sp.shape, lambda: (0, 0)),
            pl.BlockSpec(b_sp2d.shape, lambda: (0, 0)),
            pl.BlockSpec(pa2d.shape, lambda: (0, 0)),
        ],
        out_specs=pl.BlockSpec((1, ro), lambda: (0, 0)),
        out_shape=jax.ShapeDtypeStruct((1, ro), F32),
    )(h0, ht, w_li, w_lh, bl4, w_sp, b_sp2d, pa2d)


def kernel(x, edge_index, edge_attr, W_proj, b_proj, W_edge, b_edge, b_conv,
           W_ih, W_hh, b_ih, b_hh, W_lstm_ih, W_lstm_hh, b_lstm_ih,
           b_lstm_hh, W_sp, b_sp, prelu_a):
    n = x.shape[0]
    e, de = edge_attr.shape
    h = W_proj.shape[0]
    d = 2 * h
    n_mp = 3
    n_s2s = 3

    quant = NW * 8 * CW
    ep = (e + quant - 1) // quant * quant
    rows = ep // CW
    src2d = jnp.concatenate(
        [edge_index[0], jnp.zeros((ep - e,), jnp.int32)]).reshape(rows, CW)
    dst2d = jnp.concatenate(
        [edge_index[1], jnp.full((ep - e,), n, jnp.int32)]).reshape(rows, CW)
    ea_pad = jnp.concatenate(
        [edge_attr, jnp.zeros((ep - e, de), F32)], axis=0)

    wk = W_edge.reshape(h, h, de).transpose(2, 0, 1).reshape(de * h, h)
    bmat = b_edge.reshape(h, h)
    s_mat = jnp.kron(jnp.eye(de, dtype=F32), jnp.ones((1, h), F32))
    t_mat = jnp.tile(jnp.eye(h, dtype=F32), (1, de))

    b_proj2d = b_proj.reshape(1, h)
    b_conv2d = b_conv.reshape(1, h)
    b_ih3 = b_ih.reshape(3, h)
    b_hh3 = b_hh.reshape(3, h)
    bl4 = (b_lstm_ih + b_lstm_hh).reshape(4, d)
    b_sp2d = b_sp.reshape(1, -1)
    pa2d = prelu_a.reshape(1, 1)
    zeros_nh = jnp.zeros((n + 8, LW), dtype=F32)

    rows_b = rows * 2 // 5 // (8 * NW) * (8 * NW)
    rows_a = rows - rows_b
    e_a = rows_a * CW
    src_a, src_b = src2d[:rows_a], src2d[rows_a:]
    dst_a, dst_b = dst2d[:rows_a], dst2d[rows_a:]
    ea_a, ea_b = ea_pad[:e_a], ea_pad[e_a:]

    h0 = _project(x, W_proj, b_proj2d, nb=2000)
    node = h0
    for _ in range(n_mp):
        hs_b, hs_a = _gather_rows(node, src_b, src_a)
        m_b = _messages(hs_b, ea_b, s_mat, t_mat, wk, bmat, eb=8192)
        m_a = _messages(hs_a, ea_a, s_mat, t_mat, wk, bmat, eb=8192)
        p_b = _scatter_add(m_b, dst_b, zeros_nh)
        p_a = _scatter_add(m_a, dst_a, zeros_nh)
        node = _gru(p_a, p_b, node, W_ih, W_hh, b_ih3, b_hh3, b_conv2d,
                    nb=2000)

    return _set2set(h0, node, W_lstm_ih, W_lstm_hh, bl4, W_sp, b_sp2d, pa2d,
                    n_s2s, h)


# speedup vs baseline: 1.0576x; 1.0576x over previous
import functools

import jax
import jax.numpy as jnp
from jax import lax
from jax.experimental import pallas as pl
from jax.experimental.pallas import tpu as pltpu
from jax.experimental.pallas import tpu_sc as plsc

F32 = jnp.float32

NC = 2
NS = 16
NW = NC * NS

LW = 128

CW = 128


def _h0_body(x_ref, w_ref, b_ref, o_ref):
    nb = x_ref.shape[0]
    h = lax.dot_general(x_ref[...], w_ref[...], (((1,), (1,)), ((), ())),
                        preferred_element_type=F32)
    h = jnp.maximum(h + b_ref[...], 0.0)
    o_ref[...] = jnp.concatenate(
        [h, jnp.zeros((nb, LW - h.shape[1]), F32)], axis=1)


def _project(x, w_proj, b_proj2d, nb):
    n, _ = x.shape
    h = w_proj.shape[0]
    grid = n // nb
    return pl.pallas_call(
        _h0_body,
        grid=(grid,),
        in_specs=[
            pl.BlockSpec((nb, x.shape[1]), lambda i: (i, 0)),
            pl.BlockSpec(w_proj.shape, lambda i: (0, 0)),
            pl.BlockSpec((1, h), lambda i: (0, 0)),
        ],
        out_specs=pl.BlockSpec((nb, LW), lambda i: (i, 0)),
        out_shape=jax.ShapeDtypeStruct((n, LW), F32),
    )(x, w_proj, b_proj2d)


def _msg_body(hs_ref, ea_ref, s_ref, t_ref, wk_ref, wb_ref, o_ref):
    h = hs_ref[:, 0:wk_ref.shape[1]]
    a = ea_ref[...]
    eb = h.shape[0]

    def dot(lhs, w):
        return lax.dot_general(lhs, w, (((1,), (0,)), ((), ())),
                               preferred_element_type=F32)

    v = dot(a, s_ref[...]) * dot(h, t_ref[...])
    m = dot(v, wk_ref[...]) + dot(h, wb_ref[...])
    o_ref[...] = jnp.concatenate(
        [m, jnp.zeros((eb, LW - m.shape[1]), F32)], axis=1)


def _messages(hsrc, edge_attr, s_mat, t_mat, w_k, w_b, eb):
    e = hsrc.shape[0]
    h = w_k.shape[1]
    grid = e // eb
    return pl.pallas_call(
        _msg_body,
        grid=(grid,),
        in_specs=[
            pl.BlockSpec((eb, LW), lambda i: (i, 0)),
            pl.BlockSpec((eb, edge_attr.shape[1]), lambda i: (i, 0)),
            pl.BlockSpec(s_mat.shape, lambda i: (0, 0)),
            pl.BlockSpec(t_mat.shape, lambda i: (0, 0)),
            pl.BlockSpec(w_k.shape, lambda i: (0, 0)),
            pl.BlockSpec(w_b.shape, lambda i: (0, 0)),
        ],
        out_specs=pl.BlockSpec((eb, LW), lambda i: (i, 0)),
        out_shape=jax.ShapeDtypeStruct((e, LW), F32),
    )(hsrc, edge_attr, s_mat, t_mat, w_k, w_b)


def _gather_rows(node, src2d):
    n = node.shape[0]
    rows, cw = src2d.shape
    ch = rows // NW
    e = rows * cw
    r_per = ((n + NS - 1) // NS + 7) // 8 * 8
    r_last = n - (NS - 1) * r_per
    mesh = plsc.VectorSubcoreMesh(core_axis_name="c", subcore_axis_name="s",
                                  num_cores=NC, num_subcores=NS)

    def body(node_hbm, src_hbm, out_hbm, shared, idx_v, row_a, row_b,
             sem_a, sem_b):
        sid = lax.axis_index("s")
        wid = sid * NC + lax.axis_index("c")
        pltpu.sync_copy(src_hbm.at[pl.ds(wid * ch, ch)], idx_v)

        @pl.when(sid < NS - 1)
        def _stage_main():
            pltpu.sync_copy(node_hbm.at[pl.ds(sid * r_per, r_per)],
                            shared.at[pl.ds(sid * r_per, r_per)])

        @pl.when(sid == NS - 1)
        def _stage_last():
            pltpu.sync_copy(node_hbm.at[pl.ds((NS - 1) * r_per, r_last)],
                            shared.at[pl.ds((NS - 1) * r_per, r_last)])

        plsc.subcore_barrier()

        def out_at(j):
            return out_hbm.at[pl.ds((wid * ch + j) * cw, cw)]

        pltpu.async_copy(shared.at[idx_v.at[0]], row_a, sem_a)

        def pair(t, carry):
            j0 = 2 * t
            pltpu.async_copy(shared.at[idx_v.at[j0 + 1]], row_b, sem_b)
            pltpu.make_async_copy(shared.at[idx_v.at[j0]], row_a,
                                  sem_a).wait()
            pltpu.sync_copy(row_a, out_at(j0))

            @pl.when(t + 1 < ch // 2)
            def _next():
                pltpu.async_copy(shared.at[idx_v.at[j0 + 2]], row_a, sem_a)

            pltpu.make_async_copy(shared.at[idx_v.at[j0 + 1]], row_b,
                                  sem_b).wait()
            pltpu.sync_copy(row_b, out_at(j0 + 1))
            return carry

        lax.fori_loop(0, ch // 2, pair, 0)

    return pl.kernel(
        body,
        out_type=jax.ShapeDtypeStruct((e, LW), F32),
        mesh=mesh,
        scratch_types=[
            pltpu.VMEM_SHARED((n, LW), F32),
            pltpu.VMEM((ch, cw), jnp.int32),
            pltpu.VMEM((cw, LW), F32),
            pltpu.VMEM((cw, LW), F32),
            pltpu.SemaphoreType.DMA,
            pltpu.SemaphoreType.DMA,
        ],
    )(node, src2d)


def _scatter_add(m, dst2d, zeros_nh):
    np_ = zeros_nh.shape[0]
    n = np_ - 8
    rows, cw = dst2d.shape
    ch = rows // NW
    r_per = ((n + NS - 1) // NS + 7) // 8 * 8
    r_last = n - (NS - 1) * r_per
    mesh = plsc.VectorSubcoreMesh(core_axis_name="c", subcore_axis_name="s",
                                  num_cores=NC, num_subcores=NS)

    def body(m_hbm, dst_hbm, z_hbm, out_hbm, shared, idx_v, row_a, row_b,
             sem_a, sem_b):
        cid = lax.axis_index("c")
        sid = lax.axis_index("s")
        wid = sid * NC + cid

        pltpu.sync_copy(dst_hbm.at[pl.ds(wid * ch, ch)], idx_v)

        @pl.when(sid < NS - 1)
        def _init_main():
            pltpu.sync_copy(z_hbm.at[pl.ds(sid * r_per, r_per)],
                            shared.at[pl.ds(sid * r_per, r_per)])

        @pl.when(sid == NS - 1)
        def _init_last():
            pltpu.sync_copy(z_hbm.at[pl.ds((NS - 1) * r_per, np_ - (NS - 1) * r_per)],
                            shared.at[pl.ds((NS - 1) * r_per, np_ - (NS - 1) * r_per)])

        plsc.subcore_barrier()

        def m_at(j):
            return m_hbm.at[pl.ds((wid * ch + j) * cw, cw)]

        pltpu.async_copy(m_at(0), row_a, sem_a)

        def pair(t, carry):
            j0 = 2 * t
            pltpu.async_copy(m_at(j0 + 1), row_b, sem_b)
            pltpu.make_async_copy(m_at(j0), row_a, sem_a).wait()
            pltpu.sync_copy(row_a, shared.at[idx_v.at[j0]], add=True)

            @pl.when(t + 1 < ch // 2)
            def _next():
                pltpu.async_copy(m_at(j0 + 2), row_a, sem_a)

            pltpu.make_async_copy(m_at(j0 + 1), row_b, sem_b).wait()
            pltpu.sync_copy(row_b, shared.at[idx_v.at[j0 + 1]], add=True)
            return carry

        lax.fori_loop(0, ch // 2, pair, 0)
        plsc.subcore_barrier()

        @pl.when(sid < NS - 1)
        def _copy_main():
            pltpu.sync_copy(shared.at[pl.ds(sid * r_per, r_per)],
                            out_hbm.at[pl.ds(cid * n + sid * r_per, r_per)])

        @pl.when(sid == NS - 1)
        def _copy_last():
            pltpu.sync_copy(
                shared.at[pl.ds((NS - 1) * r_per, r_last)],
                out_hbm.at[pl.ds(cid * n + (NS - 1) * r_per, r_last)])

    return pl.kernel(
        body,
        out_type=jax.ShapeDtypeStruct((2 * n, LW), F32),
        mesh=mesh,
        scratch_types=[
            pltpu.VMEM_SHARED((np_, LW), F32),
            pltpu.VMEM((ch, cw), jnp.int32),
            pltpu.VMEM((cw, LW), F32),
            pltpu.VMEM((cw, LW), F32),
            pltpu.SemaphoreType.DMA,
            pltpu.SemaphoreType.DMA,
        ],
    )(m, dst2d, zeros_nh)


def _gru_body(p0_ref, p1_ref, p2_ref, p3_ref, h_ref, wih_ref, whh_ref,
              bih_ref, bhh_ref, bc_ref, o_ref, *, h_dim):
    nb = h_ref.shape[0]
    hdim = h_dim
    a = jnp.maximum(p0_ref[:, 0:hdim] + p1_ref[:, 0:hdim] +
                    p2_ref[:, 0:hdim] + p3_ref[:, 0:hdim] + bc_ref[...], 0.0)
    hprev = h_ref[:, 0:hdim]

    def dot_t(lhs, w):
        return lax.dot_general(lhs, w, (((1,), (1,)), ((), ())),
                               preferred_element_type=F32)

    wih = wih_ref[...]
    whh = whh_ref[...]
    i_r = dot_t(a, wih[0:hdim]) + bih_ref[0:1]
    i_z = dot_t(a, wih[hdim:2 * hdim]) + bih_ref[1:2]
    i_n = dot_t(a, wih[2 * hdim:3 * hdim]) + bih_ref[2:3]
    h_r = dot_t(hprev, whh[0:hdim]) + bhh_ref[0:1]
    h_z = dot_t(hprev, whh[hdim:2 * hdim]) + bhh_ref[1:2]
    h_n = dot_t(hprev, whh[2 * hdim:3 * hdim]) + bhh_ref[2:3]
    r = jax.nn.sigmoid(i_r + h_r)
    z = jax.nn.sigmoid(i_z + h_z)
    nn = jnp.tanh(i_n + r * h_n)
    hnew = (1.0 - z) * nn + z * hprev
    o_ref[...] = jnp.concatenate(
        [hnew, jnp.zeros((nb, LW - hdim), F32)], axis=1)


def _gru(parts_a, parts_b, hidden, w_ih, w_hh, b_ih3, b_hh3, b_conv2d, nb):
    n = hidden.shape[0]
    h = w_ih.shape[1]
    grid = n // nb
    return pl.pallas_call(
        functools.partial(_gru_body, h_dim=h),
        grid=(grid,),
        in_specs=[
            pl.BlockSpec((nb, LW), lambda i: (i, 0)),
            pl.BlockSpec((nb, LW), lambda i, g=grid: (i + g, 0)),
            pl.BlockSpec((nb, LW), lambda i: (i, 0)),
            pl.BlockSpec((nb, LW), lambda i, g=grid: (i + g, 0)),
            pl.BlockSpec((nb, LW), lambda i: (i, 0)),
            pl.BlockSpec(w_ih.shape, lambda i: (0, 0)),
            pl.BlockSpec(w_hh.shape, lambda i: (0, 0)),
            pl.BlockSpec((3, h), lambda i: (0, 0)),
            pl.BlockSpec((3, h), lambda i: (0, 0)),
            pl.BlockSpec((1, h), lambda i: (0, 0)),
        ],
        out_specs=pl.BlockSpec((nb, LW), lambda i: (i, 0)),
        out_shape=jax.ShapeDtypeStruct((n, LW), F32),
    )(parts_a, parts_a, parts_b, parts_b, hidden, w_ih, w_hh, b_ih3, b_hh3,
      b_conv2d)


def _s2s_body(h0_ref, ht_ref, wli_ref, wlh_ref, bl_ref, wsp_ref, bsp_ref,
              pa_ref, o_ref, *, n_s2s, h_dim):
    na = jnp.concatenate([h0_ref[:, 0:h_dim], ht_ref[:, 0:h_dim]], axis=1)
    d = na.shape[1]
    wli = wli_ref[...]
    wlh = wlh_ref[...]
    bl = bl_ref[...]
    hs = jnp.zeros((1, d), dtype=F32)
    cs = jnp.zeros((1, d), dtype=F32)
    qs = jnp.zeros((1, 2 * d), dtype=F32)

    def dot_t(lhs, w):
        return lax.dot_general(lhs, w, (((1,), (1,)), ((), ())),
                               preferred_element_type=F32)

    for _ in range(n_s2s):
        gi = dot_t(qs, wli[0:d]) + dot_t(hs, wlh[0:d]) + bl[0:1]
        gf = dot_t(qs, wli[d:2 * d]) + dot_t(hs, wlh[d:2 * d]) + bl[1:2]
        gg = dot_t(qs, wli[2 * d:3 * d]) + dot_t(hs, wlh[2 * d:3 * d]) + bl[2:3]
        go = dot_t(qs, wli[3 * d:4 * d]) + dot_t(hs, wlh[3 * d:4 * d]) + bl[3:4]
        ig = jax.nn.sigmoid(gi)
        fg = jax.nn.sigmoid(gf)
        gg = jnp.tanh(gg)
        og = jax.nn.sigmoid(go)
        cs = fg * cs + ig * gg
        hs = og * jnp.tanh(cs)
        e = dot_t(na, hs)
        mx = jnp.max(e)
        ex = jnp.exp(e - mx)
        alpha = ex / jnp.sum(ex)
        r = lax.dot_general(alpha, na, (((0,), (0,)), ((), ())),
                            preferred_element_type=F32)
        qs = jnp.concatenate([hs, r], axis=1)

    out = dot_t(qs, wsp_ref[...]) + bsp_ref[...]
    o_ref[...] = jnp.where(out >= 0.0, out, pa_ref[...] * out)


def _set2set(h0, ht, w_li, w_lh, bl4, w_sp, b_sp2d, pa2d, n_s2s, h):
    n = h0.shape[0]
    ro = w_sp.shape[0]
    return pl.pallas_call(
        functools.partial(_s2s_body, n_s2s=n_s2s, h_dim=h),
        in_specs=[
            pl.BlockSpec((n, LW), lambda: (0, 0)),
            pl.BlockSpec((n, LW), lambda: (0, 0)),
            pl.BlockSpec(w_li.shape, lambda: (0, 0)),
            pl.BlockSpec(w_lh.shape, lambda: (0, 0)),
            pl.BlockSpec(bl4.shape, lambda: (0, 0)),
            pl.BlockSpec(w_sp.shape, lambda: (0, 0)),
            pl.BlockSpec(b_sp2d.shape, lambda: (0, 0)),
            pl.BlockSpec(pa2d.shape, lambda: (0, 0)),
        ],
        out_specs=pl.BlockSpec((1, ro), lambda: (0, 0)),
        out_shape=jax.ShapeDtypeStruct((1, ro), F32),
    )(h0, ht, w_li, w_lh, bl4, w_sp, b_sp2d, pa2d)


def kernel(x, edge_index, edge_attr, W_proj, b_proj, W_edge, b_edge, b_conv,
           W_ih, W_hh, b_ih, b_hh, W_lstm_ih, W_lstm_hh, b_lstm_ih,
           b_lstm_hh, W_sp, b_sp, prelu_a):
    n = x.shape[0]
    e, de = edge_attr.shape
    h = W_proj.shape[0]
    d = 2 * h
    n_mp = 3
    n_s2s = 3

    quant = NW * 8 * CW
    ep = (e + quant - 1) // quant * quant
    rows = ep // CW
    src2d = jnp.concatenate(
        [edge_index[0], jnp.zeros((ep - e,), jnp.int32)]).reshape(rows, CW)
    dst2d = jnp.concatenate(
        [edge_index[1], jnp.full((ep - e,), n, jnp.int32)]).reshape(rows, CW)
    ea_pad = jnp.concatenate(
        [edge_attr, jnp.zeros((ep - e, de), F32)], axis=0)

    wk = W_edge.reshape(h, h, de).transpose(2, 0, 1).reshape(de * h, h)
    bmat = b_edge.reshape(h, h)
    s_mat = jnp.kron(jnp.eye(de, dtype=F32), jnp.ones((1, h), F32))
    t_mat = jnp.tile(jnp.eye(h, dtype=F32), (1, de))

    b_proj2d = b_proj.reshape(1, h)
    b_conv2d = b_conv.reshape(1, h)
    b_ih3 = b_ih.reshape(3, h)
    b_hh3 = b_hh.reshape(3, h)
    bl4 = (b_lstm_ih + b_lstm_hh).reshape(4, d)
    b_sp2d = b_sp.reshape(1, -1)
    pa2d = prelu_a.reshape(1, 1)
    zeros_nh = jnp.zeros((n + 8, LW), dtype=F32)

    rows_b = rows * 2 // 5 // (8 * NW) * (8 * NW)
    rows_a = rows - rows_b
    e_a = rows_a * CW
    src_a, src_b = src2d[:rows_a], src2d[rows_a:]
    dst_a, dst_b = dst2d[:rows_a], dst2d[rows_a:]
    ea_a, ea_b = ea_pad[:e_a], ea_pad[e_a:]

    h0 = _project(x, W_proj, b_proj2d, nb=2000)
    node = h0
    for _ in range(n_mp):
        hs_b = _gather_rows(node, src_b)
        hs_a = _gather_rows(node, src_a)
        m_b = _messages(hs_b, ea_b, s_mat, t_mat, wk, bmat, eb=8192)
        m_a = _messages(hs_a, ea_a, s_mat, t_mat, wk, bmat, eb=8192)
        p_b = _scatter_add(m_b, dst_b, zeros_nh)
        p_a = _scatter_add(m_a, dst_a, zeros_nh)
        node = _gru(p_a, p_b, node, W_ih, W_hh, b_ih3, b_hh3, b_conv2d,
                    nb=2000)

    return _set2set(h0, node, W_lstm_ih, W_lstm_hh, bl4, W_sp, b_sp2d, pa2d,
                    n_s2s, h)
